# baseline (speedup 1.0000x reference)
import functools

import jax
import jax.numpy as jnp
from jax import lax
from jax.experimental import pallas as pl
from jax.experimental.pallas import tpu as pltpu
from jax.experimental.pallas import tpu_sc as plsc

B = 1024
D = 128
K = 32
BLK = 128
NBLK = B // BLK

NC = 2
NS = 16
NW = NC * NS
QPW = B // NW
L = 16
SENT = 1 << 20


def _i32(x):
    x &= 0xFFFFFFFF
    return jnp.int32(x - (1 << 32) if x >= (1 << 31) else x)


def _select_kernel(m_full_ref, m_blk_ref, selidx_ref, sut_ref):
    i = pl.program_id(0)

    @pl.when(i == 0)
    def _():
        r_io = lax.broadcasted_iota(jnp.int32, (B, B), 0)
        c_io = lax.broadcasted_iota(jnp.int32, (B, B), 1)
        sut_ref[...] = jnp.where(r_io < c_io, 1.0, 0.0).astype(jnp.bfloat16)

    m_full = m_full_ref[...]
    m_blk = m_blk_ref[...]
    m2_full = m_full * m_full

    dot = functools.partial(
        lax.dot_general,
        preferred_element_type=jnp.float32,
        precision=lax.Precision.HIGHEST,
    )

    ones_row = jnp.ones((1, D), dtype=jnp.float32)
    rn_row = dot(ones_row, m2_full, (((1,), (1,)), ((), ())))
    g = dot(m_blk, m_full, (((1,), (1,)), ((), ())))

    score = rn_row - 2.0 * g

    col = lax.broadcasted_iota(jnp.int32, (BLK, B), 1)
    row_g = lax.broadcasted_iota(jnp.int32, (BLK, B), 0) + i * BLK

    bits = lax.bitcast_convert_type(score, jnp.int32)
    mono = bits ^ (lax.shift_right_arithmetic(bits, 31) & jnp.int32(0x7FFFFFFF))
    imax = jnp.int32(0x7FFFFFFF)
    keys = (mono & jnp.int32(~1023)) | col
    keys = jnp.where(col == row_g, imax, keys)

    isign = jnp.int32(-(2**31))
    w = keys ^ isign
    ones_b1 = jnp.ones((B, 1), dtype=jnp.bfloat16)
    p = jnp.zeros((BLK, 1), dtype=jnp.int32)
    need = jnp.full((BLK, 1), float(K), dtype=jnp.float32)
    for j in range(16):
        b0 = 30 - 2 * j
        mhi = _i32(0xFFFFFFFF << b0)
        wp = w & mhi
        eq0 = wp == p
        eq1 = wp == (p | _i32(1 << b0))
        eq2 = wp == (p | _i32(2 << b0))
        ind_a = (jnp.where(eq0, 1.0, 0.0)
                 + jnp.where(eq1, 4096.0, 0.0)).astype(jnp.bfloat16)
        ind_b = jnp.where(eq2, 1.0, 0.0).astype(jnp.bfloat16)
        dot_cnt = functools.partial(
            lax.dot_general,
            dimension_numbers=(((1,), (0,)), ((), ())),
            preferred_element_type=jnp.float32,
        )
        c_a = dot_cnt(ind_a, ones_b1)
        c2 = dot_cnt(ind_b, ones_b1)
        c1 = jnp.floor(c_a * (1.0 / 4096.0))
        c0 = c_a - 4096.0 * c1
        t01 = c0 + c1
        t012 = t01 + c2
        ge1 = need > c0
        ge2 = need > t01
        ge3 = need > t012
        digit = (
            jnp.where(ge1, jnp.int32(1), jnp.int32(0))
            + jnp.where(ge2, jnp.int32(1), jnp.int32(0))
            + jnp.where(ge3, jnp.int32(1), jnp.int32(0))
        )
        p = p | (digit * _i32(1 << b0))
        need = (
            need
            - jnp.where(ge1, c0, 0.0)
            - jnp.where(ge2, c1, 0.0)
            - jnp.where(ge3, c2, 0.0)
        )

    thr = p ^ isign
    le = keys <= thr
    sel_bf = jnp.where(le, 1.0, 0.0).astype(jnp.bfloat16)
    ranks = lax.dot_general(
        sel_bf, sut_ref[...], (((1,), (0,)), ((), ())),
        preferred_element_type=jnp.float32,
    ).astype(jnp.int32)
    packed = lax.shift_left(col, 6) | ranks
    selidx_ref[...] = jnp.where(le, packed, jnp.int32(SENT))


def _run_select(m):
    return pl.pallas_call(
        _select_kernel,
        grid=(NBLK,),
        in_specs=[
            pl.BlockSpec((B, D), lambda i: (0, 0)),
            pl.BlockSpec((BLK, D), lambda i: (i, 0)),
        ],
        out_specs=pl.BlockSpec((BLK, B), lambda i: (i, 0)),
        out_shape=jax.ShapeDtypeStruct((B, B), jnp.int32),
        scratch_shapes=[pltpu.VMEM((B, B), jnp.bfloat16)],
    )(m, m)


def _sc_gather_kernel(selidx_hbm, m_hbm, s1_hbm, s2_hbm,
                      sel_v, idx_all, buf0, buf1, s1_v, s2_v, sem0, sem1):
    wid = lax.axis_index("s") * NC + lax.axis_index("c")
    base = wid * QPW

    pltpu.sync_copy(selidx_hbm.at[pl.ds(base, QPW)], sel_v)

    lane = lax.iota(jnp.int32, L)

    def compact_row(q, _):
        qv = jnp.full((L,), q, dtype=jnp.int32)

        def compact_vreg(j, _):
            v = sel_v[q, pl.ds(j * L, L)]
            mask = v < jnp.int32(1 << 16)
            pos = jnp.where(mask, v & jnp.int32(63), K + lane)
            cols = lax.shift_right_logical(v, 6)
            plsc.store_scatter(idx_all, [qv, pos], cols)
            return 0

        lax.fori_loop(0, B // L, compact_vreg, 0)
        return 0

    lax.fori_loop(0, QPW, compact_row, 0)

    def accumulate(buf, q):
        for c in range(D // L):
            sl = pl.ds(c * L, L)
            acc_s = jnp.zeros((L,), dtype=jnp.float32)
            acc_q = jnp.zeros((L,), dtype=jnp.float32)

            def nb_body(nb, carry):
                a_s, a_q = carry
                v = buf[nb, sl]
                return a_s + v, a_q + v * v

            acc_s, acc_q = lax.fori_loop(0, K, nb_body, (acc_s, acc_q))
            s1_v[q, sl] = acc_s
            s2_v[q, sl] = acc_q

    def idx_row(q):
        return idx_all.at[q, pl.ds(0, K)]

    pltpu.async_copy(m_hbm.at[idx_row(0)], buf0, sem0)

    def pair_body(h, _):
        q0 = h * 2
        cp1 = pltpu.async_copy(m_hbm.at[idx_row(q0 + 1)], buf1, sem1)
        pltpu.make_async_copy(m_hbm.at[idx_row(0)], buf0, sem0).wait()
        accumulate(buf0, q0)

        @pl.when(q0 + 2 < QPW)
        def _():
            pltpu.async_copy(m_hbm.at[idx_row(q0 + 2)], buf0, sem0)

        cp1.wait()
        accumulate(buf1, q0 + 1)
        return 0

    lax.fori_loop(0, QPW // 2, pair_body, 0)

    pltpu.sync_copy(s1_v, s1_hbm.at[pl.ds(base, QPW)])
    pltpu.sync_copy(s2_v, s2_hbm.at[pl.ds(base, QPW)])


def _run_sc_gather(selidx, m):
    mesh = plsc.VectorSubcoreMesh(core_axis_name="c", subcore_axis_name="s")
    f = pl.kernel(
        _sc_gather_kernel,
        mesh=mesh,
        compiler_params=pltpu.CompilerParams(needs_layout_passes=False),
        out_type=(
            jax.ShapeDtypeStruct((B, D), jnp.float32),
            jax.ShapeDtypeStruct((B, D), jnp.float32),
        ),
        scratch_types=[
            pltpu.VMEM((QPW, B), jnp.int32),
            pltpu.VMEM((QPW, K + L), jnp.int32),
            pltpu.VMEM((K, D), jnp.float32),
            pltpu.VMEM((K, D), jnp.float32),
            pltpu.VMEM((QPW, D), jnp.float32),
            pltpu.VMEM((QPW, D), jnp.float32),
            pltpu.SemaphoreType.DMA,
            pltpu.SemaphoreType.DMA,
        ],
    )
    return f(selidx, m)


def _loss_kernel(s1_ref, s2_ref, v_ref, out_ref):
    s1 = s1_ref[...]
    s2 = s2_ref[...]
    var = (s2 - s1 * s1 * (1.0 / K)) * (1.0 / (K - 1))
    nstd = jnp.sqrt(jnp.maximum(var, 0.0))
    pstd = jnp.exp(v_ref[...] * 0.5)
    out_ref[0, 0] = jnp.sum((pstd - nstd) ** 2)


def _run_loss(s1, s2, v):
    return pl.pallas_call(
        _loss_kernel,
        out_specs=pl.BlockSpec(memory_space=pltpu.SMEM),
        out_shape=jax.ShapeDtypeStruct((1, 1), jnp.float32),
    )(s1, s2, v)


def kernel(input1_mean, input1_var, input2_mean, input2_var):
    del input2_mean, input2_var
    selidx = _run_select(input1_mean)
    s1, s2 = _run_sc_gather(selidx, input1_mean)
    out = _run_loss(s1, s2, input1_var)
    return (out[0, 0] * (1.0 / (B * D))).reshape(())


# speedup vs baseline: 1.0985x; 1.0985x over previous
import functools

import jax
import jax.numpy as jnp
from jax import lax
from jax.experimental import pallas as pl
from jax.experimental.pallas import tpu as pltpu
from jax.experimental.pallas import tpu_sc as plsc

B = 1024
D = 128
K = 32
BLK = 128
NBLK = B // BLK

NC = 2
NS = 16
NW = NC * NS
QPW = B // NW
L = 16
SENT = 1 << 20


def _i32(x):
    x &= 0xFFFFFFFF
    return jnp.int32(x - (1 << 32) if x >= (1 << 31) else x)


def _select_kernel(m_full_ref, m_blk_ref, selidx_ref, sut_ref):
    i = pl.program_id(0)

    @pl.when(i == 0)
    def _():
        r_io = lax.broadcasted_iota(jnp.int32, (B, B), 0)
        c_io = lax.broadcasted_iota(jnp.int32, (B, B), 1)
        sut_ref[...] = jnp.where(r_io < c_io, 1.0, 0.0).astype(jnp.bfloat16)

    m_full = m_full_ref[...]
    m_blk = m_blk_ref[...]
    m2_full = m_full * m_full

    dot = functools.partial(
        lax.dot_general,
        preferred_element_type=jnp.float32,
        precision=lax.Precision.HIGHEST,
    )

    ones_row = jnp.ones((1, D), dtype=jnp.float32)
    rn_row = dot(ones_row, m2_full, (((1,), (1,)), ((), ())))
    g = dot(m_blk, m_full, (((1,), (1,)), ((), ())))

    score = rn_row - 2.0 * g

    col = lax.broadcasted_iota(jnp.int32, (BLK, B), 1)
    row_g = lax.broadcasted_iota(jnp.int32, (BLK, B), 0) + i * BLK

    bits = lax.bitcast_convert_type(score, jnp.int32)
    mono = bits ^ (lax.shift_right_arithmetic(bits, 31) & jnp.int32(0x7FFFFFFF))
    imax = jnp.int32(0x7FFFFFFF)
    keys = (mono & jnp.int32(~1023)) | col
    keys = jnp.where(col == row_g, imax, keys)

    isign = jnp.int32(-(2**31))
    w = keys ^ isign
    ones_b1 = jnp.ones((B, 1), dtype=jnp.bfloat16)
    p = jnp.zeros((BLK, 1), dtype=jnp.int32)
    need = jnp.full((BLK, 1), float(K), dtype=jnp.float32)
    for j in range(16):
        b0 = 30 - 2 * j
        mhi = _i32(0xFFFFFFFF << b0)
        wp = w & mhi
        eq0 = wp == p
        eq1 = wp == (p | _i32(1 << b0))
        eq2 = wp == (p | _i32(2 << b0))
        ind_a = (jnp.where(eq0, 1.0, 0.0)
                 + jnp.where(eq1, 4096.0, 0.0)).astype(jnp.bfloat16)
        ind_b = jnp.where(eq2, 1.0, 0.0).astype(jnp.bfloat16)
        dot_cnt = functools.partial(
            lax.dot_general,
            dimension_numbers=(((1,), (0,)), ((), ())),
            preferred_element_type=jnp.float32,
        )
        c_a = dot_cnt(ind_a, ones_b1)
        c2 = dot_cnt(ind_b, ones_b1)
        c1 = jnp.floor(c_a * (1.0 / 4096.0))
        c0 = c_a - 4096.0 * c1
        t01 = c0 + c1
        t012 = t01 + c2
        ge1 = need > c0
        ge2 = need > t01
        ge3 = need > t012
        digit = (
            jnp.where(ge1, jnp.int32(1), jnp.int32(0))
            + jnp.where(ge2, jnp.int32(1), jnp.int32(0))
            + jnp.where(ge3, jnp.int32(1), jnp.int32(0))
        )
        p = p | (digit * _i32(1 << b0))
        need = (
            need
            - jnp.where(ge1, c0, 0.0)
            - jnp.where(ge2, c1, 0.0)
            - jnp.where(ge3, c2, 0.0)
        )

    thr = p ^ isign
    le = keys <= thr
    sel_bf = jnp.where(le, 1.0, 0.0).astype(jnp.bfloat16)
    ranks = lax.dot_general(
        sel_bf, sut_ref[...], (((1,), (0,)), ((), ())),
        preferred_element_type=jnp.float32,
    ).astype(jnp.int32)
    packed = lax.shift_left(col, 6) | ranks
    selidx_ref[...] = jnp.where(le, packed, jnp.int32(SENT))


def _run_select(m):
    return pl.pallas_call(
        _select_kernel,
        grid=(NBLK,),
        in_specs=[
            pl.BlockSpec((B, D), lambda i: (0, 0)),
            pl.BlockSpec((BLK, D), lambda i: (i, 0)),
        ],
        out_specs=pl.BlockSpec((BLK, B), lambda i: (i, 0)),
        out_shape=jax.ShapeDtypeStruct((B, B), jnp.int32),
        scratch_shapes=[pltpu.VMEM((B, B), jnp.bfloat16)],
    )(m, m)


def _sc_gather_kernel(selidx_hbm, m_hbm, s1_hbm, s2_hbm,
                      sel_v, idx_all, buf0, buf1, s1_v, s2_v, sem0, sem1):
    wid = lax.axis_index("s") * NC + lax.axis_index("c")
    base = wid * QPW

    pltpu.sync_copy(selidx_hbm.at[pl.ds(base, QPW)], sel_v)

    lane = lax.iota(jnp.int32, L)

    def compact_row(q, _):
        qv = jnp.full((L,), q, dtype=jnp.int32)
        for j in range(B // L):
            v = sel_v[q, pl.ds(j * L, L)]
            mask = v < jnp.int32(1 << 16)
            pos = jnp.where(mask, v & jnp.int32(63), K + lane)
            cols = lax.shift_right_logical(v, 6)
            plsc.store_scatter(idx_all, [qv, pos], cols)
        return 0

    lax.fori_loop(0, QPW, compact_row, 0)

    nch = D // L

    def accumulate(buf, q):
        def nb_body(nb, carry):
            a_s, a_q = carry
            new_s, new_q = [], []
            for c in range(nch):
                for u in range(4):
                    v = buf[nb * 4 + u, pl.ds(c * L, L)]
                    a_s_c = (a_s[c] + v) if u == 0 else (new_s[c] + v)
                    a_q_c = (a_q[c] + v * v) if u == 0 else (new_q[c] + v * v)
                    if u == 0:
                        new_s.append(a_s_c)
                        new_q.append(a_q_c)
                    else:
                        new_s[c] = a_s_c
                        new_q[c] = a_q_c
            return tuple(new_s), tuple(new_q)

        zeros = tuple(jnp.zeros((L,), dtype=jnp.float32) for _ in range(nch))
        acc_s, acc_q = lax.fori_loop(0, K // 4, nb_body, (zeros, zeros))
        for c in range(nch):
            sl = pl.ds(c * L, L)
            s1_v[q, sl] = acc_s[c]
            s2_v[q, sl] = acc_q[c]

    def idx_row(q):
        return idx_all.at[q, pl.ds(0, K)]

    pltpu.async_copy(m_hbm.at[idx_row(0)], buf0, sem0)

    def pair_body(h, _):
        q0 = h * 2
        cp1 = pltpu.async_copy(m_hbm.at[idx_row(q0 + 1)], buf1, sem1)
        pltpu.make_async_copy(m_hbm.at[idx_row(0)], buf0, sem0).wait()
        accumulate(buf0, q0)

        @pl.when(q0 + 2 < QPW)
        def _():
            pltpu.async_copy(m_hbm.at[idx_row(q0 + 2)], buf0, sem0)

        cp1.wait()
        accumulate(buf1, q0 + 1)
        return 0

    lax.fori_loop(0, QPW // 2, pair_body, 0)

    pltpu.sync_copy(s1_v, s1_hbm.at[pl.ds(base, QPW)])
    pltpu.sync_copy(s2_v, s2_hbm.at[pl.ds(base, QPW)])


def _run_sc_gather(selidx, m):
    mesh = plsc.VectorSubcoreMesh(core_axis_name="c", subcore_axis_name="s")
    f = pl.kernel(
        _sc_gather_kernel,
        mesh=mesh,
        compiler_params=pltpu.CompilerParams(needs_layout_passes=False),
        out_type=(
            jax.ShapeDtypeStruct((B, D), jnp.float32),
            jax.ShapeDtypeStruct((B, D), jnp.float32),
        ),
        scratch_types=[
            pltpu.VMEM((QPW, B), jnp.int32),
            pltpu.VMEM((QPW, K + L), jnp.int32),
            pltpu.VMEM((K, D), jnp.float32),
            pltpu.VMEM((K, D), jnp.float32),
            pltpu.VMEM((QPW, D), jnp.float32),
            pltpu.VMEM((QPW, D), jnp.float32),
            pltpu.SemaphoreType.DMA,
            pltpu.SemaphoreType.DMA,
        ],
    )
    return f(selidx, m)


def _loss_kernel(s1_ref, s2_ref, v_ref, out_ref):
    s1 = s1_ref[...]
    s2 = s2_ref[...]
    var = (s2 - s1 * s1 * (1.0 / K)) * (1.0 / (K - 1))
    nstd = jnp.sqrt(jnp.maximum(var, 0.0))
    pstd = jnp.exp(v_ref[...] * 0.5)
    out_ref[0, 0] = jnp.sum((pstd - nstd) ** 2)


def _run_loss(s1, s2, v):
    return pl.pallas_call(
        _loss_kernel,
        out_specs=pl.BlockSpec(memory_space=pltpu.SMEM),
        out_shape=jax.ShapeDtypeStruct((1, 1), jnp.float32),
    )(s1, s2, v)


def kernel(input1_mean, input1_var, input2_mean, input2_var):
    del input2_mean, input2_var
    selidx = _run_select(input1_mean)
    s1, s2 = _run_sc_gather(selidx, input1_mean)
    out = _run_loss(s1, s2, input1_var)
    return (out[0, 0] * (1.0 / (B * D))).reshape(())


# speedup vs baseline: 1.1344x; 1.0327x over previous
import functools

import jax
import jax.numpy as jnp
from jax import lax
from jax.experimental import pallas as pl
from jax.experimental.pallas import tpu as pltpu
from jax.experimental.pallas import tpu_sc as plsc

B = 1024
D = 128
K = 32
BLK = 128
NBLK = B // BLK

NC = 2
NS = 16
NW = NC * NS
QPW = B // NW
L = 16
SENT = 1 << 20


def _i32(x):
    x &= 0xFFFFFFFF
    return jnp.int32(x - (1 << 32) if x >= (1 << 31) else x)


def _select_kernel(m_full_ref, m_blk_ref, selidx_ref, sut_ref):
    i = pl.program_id(0)

    @pl.when(i == 0)
    def _():
        r_io = lax.broadcasted_iota(jnp.int32, (B, B), 0)
        c_io = lax.broadcasted_iota(jnp.int32, (B, B), 1)
        sut_ref[...] = jnp.where(r_io < c_io, 1.0, 0.0).astype(jnp.bfloat16)

    m_full = m_full_ref[...]
    m_blk = m_blk_ref[...]
    m2_full = m_full * m_full

    dot = functools.partial(
        lax.dot_general,
        preferred_element_type=jnp.float32,
        precision=lax.Precision.HIGHEST,
    )

    ones_row = jnp.ones((1, D), dtype=jnp.float32)
    rn_row = dot(ones_row, m2_full, (((1,), (1,)), ((), ())))
    g = dot(m_blk, m_full, (((1,), (1,)), ((), ())))

    score = rn_row - 2.0 * g

    col = lax.broadcasted_iota(jnp.int32, (BLK, B), 1)
    row_g = lax.broadcasted_iota(jnp.int32, (BLK, B), 0) + i * BLK

    bits = lax.bitcast_convert_type(score, jnp.int32)
    mono = bits ^ (lax.shift_right_arithmetic(bits, 31) & jnp.int32(0x7FFFFFFF))
    imax = jnp.int32(0x7FFFFFFF)
    keys = (mono & jnp.int32(~1023)) | col
    keys = jnp.where(col == row_g, imax, keys)

    isign = jnp.int32(-(2**31))
    w = keys ^ isign
    ones_b1 = jnp.ones((B, 1), dtype=jnp.bfloat16)
    dot_cnt = functools.partial(
        lax.dot_general,
        dimension_numbers=(((1,), (0,)), ((), ())),
        preferred_element_type=jnp.float32,
    )
    H = 2
    HR = BLK // H
    ws = [lax.slice(w, (h * HR, 0), ((h + 1) * HR, B)) for h in range(H)]
    ps = [jnp.zeros((HR, 1), dtype=jnp.int32) for _ in range(H)]
    needs = [jnp.full((HR, 1), float(K), dtype=jnp.float32) for _ in range(H)]
    for j in range(16):
        b0 = 30 - 2 * j
        mhi = _i32(0xFFFFFFFF << b0)
        cas, c2s = [], []
        for h in range(H):
            wp = ws[h] & mhi
            eq0 = wp == ps[h]
            eq1 = wp == (ps[h] | _i32(1 << b0))
            eq2 = wp == (ps[h] | _i32(2 << b0))
            ind_a = (jnp.where(eq0, 1.0, 0.0)
                     + jnp.where(eq1, 4096.0, 0.0)).astype(jnp.bfloat16)
            ind_b = jnp.where(eq2, 1.0, 0.0).astype(jnp.bfloat16)
            cas.append(dot_cnt(ind_a, ones_b1))
            c2s.append(dot_cnt(ind_b, ones_b1))
        for h in range(H):
            c_a, c2 = cas[h], c2s[h]
            need = needs[h]
            c1 = jnp.floor(c_a * (1.0 / 4096.0))
            c0 = c_a - 4096.0 * c1
            t01 = c0 + c1
            t012 = t01 + c2
            ge1 = need > c0
            ge2 = need > t01
            ge3 = need > t012
            digit = (
                jnp.where(ge1, jnp.int32(1), jnp.int32(0))
                + jnp.where(ge2, jnp.int32(1), jnp.int32(0))
                + jnp.where(ge3, jnp.int32(1), jnp.int32(0))
            )
            ps[h] = ps[h] | (digit * _i32(1 << b0))
            needs[h] = (
                need
                - jnp.where(ge1, c0, 0.0)
                - jnp.where(ge2, c1, 0.0)
                - jnp.where(ge3, c2, 0.0)
            )

    thr = jnp.concatenate(ps, axis=0) ^ isign
    le = keys <= thr
    sel_bf = jnp.where(le, 1.0, 0.0).astype(jnp.bfloat16)
    ranks = lax.dot_general(
        sel_bf, sut_ref[...], (((1,), (0,)), ((), ())),
        preferred_element_type=jnp.float32,
    ).astype(jnp.int32)
    packed = lax.shift_left(col, 6) | ranks
    selidx_ref[...] = jnp.where(le, packed, jnp.int32(SENT))


def _run_select(m):
    return pl.pallas_call(
        _select_kernel,
        grid=(NBLK,),
        in_specs=[
            pl.BlockSpec((B, D), lambda i: (0, 0)),
            pl.BlockSpec((BLK, D), lambda i: (i, 0)),
        ],
        out_specs=pl.BlockSpec((BLK, B), lambda i: (i, 0)),
        out_shape=jax.ShapeDtypeStruct((B, B), jnp.int32),
        scratch_shapes=[pltpu.VMEM((B, B), jnp.bfloat16)],
    )(m, m)


def _sc_gather_kernel(selidx_hbm, m_hbm, s1_hbm, s2_hbm,
                      sel_v, idx_all, buf0, buf1, s1_v, s2_v, sem0, sem1):
    wid = lax.axis_index("s") * NC + lax.axis_index("c")
    base = wid * QPW

    pltpu.sync_copy(selidx_hbm.at[pl.ds(base, QPW)], sel_v)

    lane = lax.iota(jnp.int32, L)

    def compact_row(q, _):
        qv = jnp.full((L,), q, dtype=jnp.int32)
        for j in range(B // L):
            v = sel_v[q, pl.ds(j * L, L)]
            mask = v < jnp.int32(1 << 16)
            pos = jnp.where(mask, v & jnp.int32(63), K + lane)
            cols = lax.shift_right_logical(v, 6)
            plsc.store_scatter(idx_all, [qv, pos], cols)
        return 0

    lax.fori_loop(0, QPW, compact_row, 0)

    nch = D // L

    def accumulate(buf, q):
        def nb_body(nb, carry):
            a_s, a_q = carry
            new_s, new_q = [], []
            for c in range(nch):
                for u in range(4):
                    v = buf[nb * 4 + u, pl.ds(c * L, L)]
                    a_s_c = (a_s[c] + v) if u == 0 else (new_s[c] + v)
                    a_q_c = (a_q[c] + v * v) if u == 0 else (new_q[c] + v * v)
                    if u == 0:
                        new_s.append(a_s_c)
                        new_q.append(a_q_c)
                    else:
                        new_s[c] = a_s_c
                        new_q[c] = a_q_c
            return tuple(new_s), tuple(new_q)

        zeros = tuple(jnp.zeros((L,), dtype=jnp.float32) for _ in range(nch))
        acc_s, acc_q = lax.fori_loop(0, K // 4, nb_body, (zeros, zeros))
        for c in range(nch):
            sl = pl.ds(c * L, L)
            s1_v[q, sl] = acc_s[c]
            s2_v[q, sl] = acc_q[c]

    def idx_row(q):
        return idx_all.at[q, pl.ds(0, K)]

    pltpu.async_copy(m_hbm.at[idx_row(0)], buf0, sem0)

    def pair_body(h, _):
        q0 = h * 2
        cp1 = pltpu.async_copy(m_hbm.at[idx_row(q0 + 1)], buf1, sem1)
        pltpu.make_async_copy(m_hbm.at[idx_row(0)], buf0, sem0).wait()
        accumulate(buf0, q0)

        @pl.when(q0 + 2 < QPW)
        def _():
            pltpu.async_copy(m_hbm.at[idx_row(q0 + 2)], buf0, sem0)

        cp1.wait()
        accumulate(buf1, q0 + 1)
        return 0

    lax.fori_loop(0, QPW // 2, pair_body, 0)

    pltpu.sync_copy(s1_v, s1_hbm.at[pl.ds(base, QPW)])
    pltpu.sync_copy(s2_v, s2_hbm.at[pl.ds(base, QPW)])


def _run_sc_gather(selidx, m):
    mesh = plsc.VectorSubcoreMesh(core_axis_name="c", subcore_axis_name="s")
    f = pl.kernel(
        _sc_gather_kernel,
        mesh=mesh,
        compiler_params=pltpu.CompilerParams(needs_layout_passes=False),
        out_type=(
            jax.ShapeDtypeStruct((B, D), jnp.float32),
            jax.ShapeDtypeStruct((B, D), jnp.float32),
        ),
        scratch_types=[
            pltpu.VMEM((QPW, B), jnp.int32),
            pltpu.VMEM((QPW, K + L), jnp.int32),
            pltpu.VMEM((K, D), jnp.float32),
            pltpu.VMEM((K, D), jnp.float32),
            pltpu.VMEM((QPW, D), jnp.float32),
            pltpu.VMEM((QPW, D), jnp.float32),
            pltpu.SemaphoreType.DMA,
            pltpu.SemaphoreType.DMA,
        ],
    )
    return f(selidx, m)


def _loss_kernel(s1_ref, s2_ref, v_ref, out_ref):
    s1 = s1_ref[...]
    s2 = s2_ref[...]
    var = (s2 - s1 * s1 * (1.0 / K)) * (1.0 / (K - 1))
    nstd = jnp.sqrt(jnp.maximum(var, 0.0))
    pstd = jnp.exp(v_ref[...] * 0.5)
    out_ref[0, 0] = jnp.sum((pstd - nstd) ** 2)


def _run_loss(s1, s2, v):
    return pl.pallas_call(
        _loss_kernel,
        out_specs=pl.BlockSpec(memory_space=pltpu.SMEM),
        out_shape=jax.ShapeDtypeStruct((1, 1), jnp.float32),
    )(s1, s2, v)


def kernel(input1_mean, input1_var, input2_mean, input2_var):
    del input2_mean, input2_var
    selidx = _run_select(input1_mean)
    s1, s2 = _run_sc_gather(selidx, input1_mean)
    out = _run_loss(s1, s2, input1_var)
    return (out[0, 0] * (1.0 / (B * D))).reshape(())


# speedup vs baseline: 1.2608x; 1.1114x over previous
import functools

import jax
import jax.numpy as jnp
from jax import lax
from jax.experimental import pallas as pl
from jax.experimental.pallas import tpu as pltpu
from jax.experimental.pallas import tpu_sc as plsc

B = 1024
D = 128
K = 32
BLK = 128
NBLK = B // BLK

NC = 2
NS = 16
NW = NC * NS
QPW = B // NW
L = 16
SENT = 1 << 20


def _i32(x):
    x &= 0xFFFFFFFF
    return jnp.int32(x - (1 << 32) if x >= (1 << 31) else x)


def _select_kernel(m_full_ref, m_blk_ref, selidx_ref, sut_ref):
    i = pl.program_id(0)

    @pl.when(i == 0)
    def _():
        r_io = lax.broadcasted_iota(jnp.int32, (B, B), 0)
        c_io = lax.broadcasted_iota(jnp.int32, (B, B), 1)
        sut_ref[...] = jnp.where(r_io < c_io, 1.0, 0.0).astype(jnp.bfloat16)

    m_full = m_full_ref[...]
    m_blk = m_blk_ref[...]
    m2_full = m_full * m_full

    dot = functools.partial(
        lax.dot_general,
        preferred_element_type=jnp.float32,
        precision=lax.Precision.HIGHEST,
    )

    ones_row = jnp.ones((1, D), dtype=jnp.float32)
    rn_row = dot(ones_row, m2_full, (((1,), (1,)), ((), ())))
    g = dot(m_blk, m_full, (((1,), (1,)), ((), ())))

    score = rn_row - 2.0 * g

    col = lax.broadcasted_iota(jnp.int32, (BLK, B), 1)
    row_g = lax.broadcasted_iota(jnp.int32, (BLK, B), 0) + i * BLK

    bits = lax.bitcast_convert_type(score, jnp.int32)
    mono = bits ^ (lax.shift_right_arithmetic(bits, 31) & jnp.int32(0x7FFFFFFF))
    imax = jnp.int32(0x7FFFFFFF)
    keys = (mono & jnp.int32(~1023)) | col
    keys = jnp.where(col == row_g, imax, keys)

    isign = jnp.int32(-(2**31))
    w = keys ^ isign
    ones_b1 = jnp.ones((B, 1), dtype=jnp.bfloat16)
    dot_cnt = functools.partial(
        lax.dot_general,
        dimension_numbers=(((1,), (0,)), ((), ())),
        preferred_element_type=jnp.float32,
    )
    H = 2
    HR = BLK // H
    ws = [lax.slice(w, (h * HR, 0), ((h + 1) * HR, B)) for h in range(H)]
    ps = [jnp.zeros((HR, 1), dtype=jnp.int32) for _ in range(H)]
    needs = [jnp.full((HR, 1), float(K), dtype=jnp.float32) for _ in range(H)]
    for j in range(16):
        b0 = 30 - 2 * j
        mhi = _i32(0xFFFFFFFF << b0)
        cas, c2s = [], []
        for h in range(H):
            wp = ws[h] & mhi
            eq0 = wp == ps[h]
            eq1 = wp == (ps[h] | _i32(1 << b0))
            eq2 = wp == (ps[h] | _i32(2 << b0))
            ind_a = (jnp.where(eq0, 1.0, 0.0)
                     + jnp.where(eq1, 4096.0, 0.0)).astype(jnp.bfloat16)
            ind_b = jnp.where(eq2, 1.0, 0.0).astype(jnp.bfloat16)
            cas.append(dot_cnt(ind_a, ones_b1))
            c2s.append(dot_cnt(ind_b, ones_b1))
        for h in range(H):
            c_a, c2 = cas[h], c2s[h]
            need = needs[h]
            c1 = jnp.floor(c_a * (1.0 / 4096.0))
            c0 = c_a - 4096.0 * c1
            t01 = c0 + c1
            t012 = t01 + c2
            ge1 = need > c0
            ge2 = need > t01
            ge3 = need > t012
            digit = (
                jnp.where(ge1, jnp.int32(1), jnp.int32(0))
                + jnp.where(ge2, jnp.int32(1), jnp.int32(0))
                + jnp.where(ge3, jnp.int32(1), jnp.int32(0))
            )
            ps[h] = ps[h] | (digit * _i32(1 << b0))
            needs[h] = (
                need
                - jnp.where(ge1, c0, 0.0)
                - jnp.where(ge2, c1, 0.0)
                - jnp.where(ge3, c2, 0.0)
            )

    thr = jnp.concatenate(ps, axis=0) ^ isign
    le = keys <= thr
    sel_bf = jnp.where(le, 1.0, 0.0).astype(jnp.bfloat16)
    ranks = lax.dot_general(
        sel_bf, sut_ref[...], (((1,), (0,)), ((), ())),
        preferred_element_type=jnp.float32,
    ).astype(jnp.int32)
    packed = lax.shift_left(col, 6) | ranks
    selidx_ref[...] = jnp.where(le, packed, jnp.int32(SENT))


def _run_select(m):
    return pl.pallas_call(
        _select_kernel,
        grid=(NBLK,),
        in_specs=[
            pl.BlockSpec((B, D), lambda i: (0, 0)),
            pl.BlockSpec((BLK, D), lambda i: (i, 0)),
        ],
        out_specs=pl.BlockSpec((BLK, B), lambda i: (i, 0)),
        out_shape=jax.ShapeDtypeStruct((B, B), jnp.int32),
        scratch_shapes=[pltpu.VMEM((B, B), jnp.bfloat16)],
    )(m, m)


def _sc_gather_kernel(selidx_hbm, m_hbm, s1_hbm, s2_hbm,
                      sel_v, idx_all, buf0, buf1, s1_v, s2_v, sem0, sem1):
    wid = lax.axis_index("s") * NC + lax.axis_index("c")
    base = wid * QPW

    pltpu.sync_copy(selidx_hbm.at[pl.ds(base, QPW)], sel_v)

    lane = lax.iota(jnp.int32, L)

    def compact_row(q):
        qv = jnp.full((L,), q, dtype=jnp.int32)
        for j in range(B // L):
            v = sel_v[q, pl.ds(j * L, L)]
            mask = v < jnp.int32(1 << 16)
            pos = jnp.where(mask, v & jnp.int32(63), K + lane)
            cols = lax.shift_right_logical(v, 6)
            plsc.store_scatter(idx_all, [qv, pos], cols)

    nch = D // L

    def accumulate(buf, q):
        def nb_body(nb, carry):
            a_s, a_q = carry
            new_s, new_q = [], []
            for c in range(nch):
                for u in range(4):
                    v = buf[nb * 4 + u, pl.ds(c * L, L)]
                    a_s_c = (a_s[c] + v) if u == 0 else (new_s[c] + v)
                    a_q_c = (a_q[c] + v * v) if u == 0 else (new_q[c] + v * v)
                    if u == 0:
                        new_s.append(a_s_c)
                        new_q.append(a_q_c)
                    else:
                        new_s[c] = a_s_c
                        new_q[c] = a_q_c
            return tuple(new_s), tuple(new_q)

        zeros = tuple(jnp.zeros((L,), dtype=jnp.float32) for _ in range(nch))
        acc_s, acc_q = lax.fori_loop(0, K // 4, nb_body, (zeros, zeros))
        for c in range(nch):
            sl = pl.ds(c * L, L)
            s1_v[q, sl] = acc_s[c]
            s2_v[q, sl] = acc_q[c]

    def idx_row(q):
        return idx_all.at[q, pl.ds(0, K)]

    compact_row(jnp.int32(0))
    compact_row(jnp.int32(1))
    pltpu.async_copy(m_hbm.at[idx_row(0)], buf0, sem0)
    pltpu.async_copy(m_hbm.at[idx_row(1)], buf1, sem1)

    def pair_body(h, _):
        q0 = h * 2

        @pl.when(q0 + 2 < QPW)
        def _():
            compact_row(q0 + 2)
            compact_row(q0 + 3)

        pltpu.make_async_copy(m_hbm.at[idx_row(0)], buf0, sem0).wait()
        accumulate(buf0, q0)

        @pl.when(q0 + 2 < QPW)
        def _():
            pltpu.async_copy(m_hbm.at[idx_row(q0 + 2)], buf0, sem0)

        pltpu.make_async_copy(m_hbm.at[idx_row(0)], buf1, sem1).wait()
        accumulate(buf1, q0 + 1)

        @pl.when(q0 + 3 < QPW)
        def _():
            pltpu.async_copy(m_hbm.at[idx_row(q0 + 3)], buf1, sem1)

        return 0

    lax.fori_loop(0, QPW // 2, pair_body, 0)

    pltpu.sync_copy(s1_v, s1_hbm.at[pl.ds(base, QPW)])
    pltpu.sync_copy(s2_v, s2_hbm.at[pl.ds(base, QPW)])


def _run_sc_gather(selidx, m):
    mesh = plsc.VectorSubcoreMesh(core_axis_name="c", subcore_axis_name="s")
    f = pl.kernel(
        _sc_gather_kernel,
        mesh=mesh,
        compiler_params=pltpu.CompilerParams(needs_layout_passes=False),
        out_type=(
            jax.ShapeDtypeStruct((B, D), jnp.float32),
            jax.ShapeDtypeStruct((B, D), jnp.float32),
        ),
        scratch_types=[
            pltpu.VMEM((QPW, B), jnp.int32),
            pltpu.VMEM((QPW, K + L), jnp.int32),
            pltpu.VMEM((K, D), jnp.float32),
            pltpu.VMEM((K, D), jnp.float32),
            pltpu.VMEM((QPW, D), jnp.float32),
            pltpu.VMEM((QPW, D), jnp.float32),
            pltpu.SemaphoreType.DMA,
            pltpu.SemaphoreType.DMA,
        ],
    )
    return f(selidx, m)


def _loss_kernel(s1_ref, s2_ref, v_ref, out_ref):
    s1 = s1_ref[...]
    s2 = s2_ref[...]
    var = (s2 - s1 * s1 * (1.0 / K)) * (1.0 / (K - 1))
    nstd = jnp.sqrt(jnp.maximum(var, 0.0))
    pstd = jnp.exp(v_ref[...] * 0.5)
    out_ref[0, 0] = jnp.sum((pstd - nstd) ** 2)


def _run_loss(s1, s2, v):
    return pl.pallas_call(
        _loss_kernel,
        out_specs=pl.BlockSpec(memory_space=pltpu.SMEM),
        out_shape=jax.ShapeDtypeStruct((1, 1), jnp.float32),
    )(s1, s2, v)


def kernel(input1_mean, input1_var, input2_mean, input2_var):
    del input2_mean, input2_var
    selidx = _run_select(input1_mean)
    s1, s2 = _run_sc_gather(selidx, input1_mean)
    out = _run_loss(s1, s2, input1_var)
    return (out[0, 0] * (1.0 / (B * D))).reshape(())


# speedup vs baseline: 1.3716x; 1.0879x over previous
import functools

import jax
import jax.numpy as jnp
from jax import lax
from jax.experimental import pallas as pl
from jax.experimental.pallas import tpu as pltpu
from jax.experimental.pallas import tpu_sc as plsc

B = 1024
D = 128
K = 32
BLK = 128
NBLK = B // BLK

NC = 2
NS = 16
NW = NC * NS
QPW = B // NW
L = 16
SENT = 1 << 20


def _i32(x):
    x &= 0xFFFFFFFF
    return jnp.int32(x - (1 << 32) if x >= (1 << 31) else x)


def _make_select_kernel(row0):
    return functools.partial(_select_kernel, row0)


def _select_kernel(row0, m_full_ref, m_blk_ref, selidx_ref, sut_ref):
    i = pl.program_id(0)

    @pl.when(i == 0)
    def _():
        r_io = lax.broadcasted_iota(jnp.int32, (B, B), 0)
        c_io = lax.broadcasted_iota(jnp.int32, (B, B), 1)
        sut_ref[...] = jnp.where(r_io < c_io, 1.0, 0.0).astype(jnp.bfloat16)

    m_full = m_full_ref[...]
    m_blk = m_blk_ref[...]
    m2_full = m_full * m_full

    dot = functools.partial(
        lax.dot_general,
        preferred_element_type=jnp.float32,
        precision=lax.Precision.HIGHEST,
    )

    ones_row = jnp.ones((1, D), dtype=jnp.float32)
    rn_row = dot(ones_row, m2_full, (((1,), (1,)), ((), ())))
    g = dot(m_blk, m_full, (((1,), (1,)), ((), ())))

    score = rn_row - 2.0 * g

    col = lax.broadcasted_iota(jnp.int32, (BLK, B), 1)
    row_g = lax.broadcasted_iota(jnp.int32, (BLK, B), 0) + i * BLK + row0

    bits = lax.bitcast_convert_type(score, jnp.int32)
    mono = bits ^ (lax.shift_right_arithmetic(bits, 31) & jnp.int32(0x7FFFFFFF))
    imax = jnp.int32(0x7FFFFFFF)
    keys = (mono & jnp.int32(~1023)) | col
    keys = jnp.where(col == row_g, imax, keys)

    isign = jnp.int32(-(2**31))
    w = keys ^ isign
    ones_b1 = jnp.ones((B, 1), dtype=jnp.bfloat16)
    dot_cnt = functools.partial(
        lax.dot_general,
        dimension_numbers=(((1,), (0,)), ((), ())),
        preferred_element_type=jnp.float32,
    )
    H = 2
    HR = BLK // H
    ws = [lax.slice(w, (h * HR, 0), ((h + 1) * HR, B)) for h in range(H)]
    ps = [jnp.zeros((HR, 1), dtype=jnp.int32) for _ in range(H)]
    needs = [jnp.full((HR, 1), float(K), dtype=jnp.float32) for _ in range(H)]
    for j in range(16):
        b0 = 30 - 2 * j
        mhi = _i32(0xFFFFFFFF << b0)
        cas, c2s = [], []
        for h in range(H):
            wp = ws[h] & mhi
            eq0 = wp == ps[h]
            eq1 = wp == (ps[h] | _i32(1 << b0))
            eq2 = wp == (ps[h] | _i32(2 << b0))
            ind_a = (jnp.where(eq0, 1.0, 0.0)
                     + jnp.where(eq1, 4096.0, 0.0)).astype(jnp.bfloat16)
            ind_b = jnp.where(eq2, 1.0, 0.0).astype(jnp.bfloat16)
            cas.append(dot_cnt(ind_a, ones_b1))
            c2s.append(dot_cnt(ind_b, ones_b1))
        for h in range(H):
            c_a, c2 = cas[h], c2s[h]
            need = needs[h]
            c1 = jnp.floor(c_a * (1.0 / 4096.0))
            c0 = c_a - 4096.0 * c1
            t01 = c0 + c1
            t012 = t01 + c2
            ge1 = need > c0
            ge2 = need > t01
            ge3 = need > t012
            digit = (
                jnp.where(ge1, jnp.int32(1), jnp.int32(0))
                + jnp.where(ge2, jnp.int32(1), jnp.int32(0))
                + jnp.where(ge3, jnp.int32(1), jnp.int32(0))
            )
            ps[h] = ps[h] | (digit * _i32(1 << b0))
            needs[h] = (
                need
                - jnp.where(ge1, c0, 0.0)
                - jnp.where(ge2, c1, 0.0)
                - jnp.where(ge3, c2, 0.0)
            )

    thr = jnp.concatenate(ps, axis=0) ^ isign
    le = keys <= thr
    sel_bf = jnp.where(le, 1.0, 0.0).astype(jnp.bfloat16)
    ranks = lax.dot_general(
        sel_bf, sut_ref[...], (((1,), (0,)), ((), ())),
        preferred_element_type=jnp.float32,
    ).astype(jnp.int32)
    packed = lax.shift_left(col, 6) | ranks
    selidx_ref[...] = jnp.where(le, packed, jnp.int32(SENT))


def _run_select(m, row0, nrows):
    blk0 = row0 // BLK
    return pl.pallas_call(
        _make_select_kernel(row0),
        grid=(nrows // BLK,),
        in_specs=[
            pl.BlockSpec((B, D), lambda i: (0, 0)),
            pl.BlockSpec((BLK, D), lambda i: (blk0 + i, 0)),
        ],
        out_specs=pl.BlockSpec((BLK, B), lambda i: (i, 0)),
        out_shape=jax.ShapeDtypeStruct((nrows, B), jnp.int32),
        scratch_shapes=[pltpu.VMEM((B, B), jnp.bfloat16)],
    )(m, m)


def _sc_gather_kernel(qpw, selidx_hbm, m_hbm, s1_hbm, s2_hbm,
                      sel_v, idx_all, buf0, buf1, s1_v, s2_v, sem0, sem1):
    wid = lax.axis_index("s") * NC + lax.axis_index("c")
    base = wid * qpw

    pltpu.sync_copy(selidx_hbm.at[pl.ds(base, qpw)], sel_v)

    lane = lax.iota(jnp.int32, L)

    def compact_row(q):
        qv = jnp.full((L,), q, dtype=jnp.int32)
        for j in range(B // L):
            v = sel_v[q, pl.ds(j * L, L)]
            mask = v < jnp.int32(1 << 16)
            pos = jnp.where(mask, v & jnp.int32(63), K + lane)
            cols = lax.shift_right_logical(v, 6)
            plsc.store_scatter(idx_all, [qv, pos], cols)

    nch = D // L

    def accumulate(buf, q):
        def nb_body(nb, carry):
            a_s, a_q = carry
            new_s, new_q = [], []
            for c in range(nch):
                for u in range(4):
                    v = buf[nb * 4 + u, pl.ds(c * L, L)]
                    a_s_c = (a_s[c] + v) if u == 0 else (new_s[c] + v)
                    a_q_c = (a_q[c] + v * v) if u == 0 else (new_q[c] + v * v)
                    if u == 0:
                        new_s.append(a_s_c)
                        new_q.append(a_q_c)
                    else:
                        new_s[c] = a_s_c
                        new_q[c] = a_q_c
            return tuple(new_s), tuple(new_q)

        zeros = tuple(jnp.zeros((L,), dtype=jnp.float32) for _ in range(nch))
        acc_s, acc_q = lax.fori_loop(0, K // 4, nb_body, (zeros, zeros))
        for c in range(nch):
            sl = pl.ds(c * L, L)
            s1_v[q, sl] = acc_s[c]
            s2_v[q, sl] = acc_q[c]

    def idx_row(q):
        return idx_all.at[q, pl.ds(0, K)]

    compact_row(jnp.int32(0))
    compact_row(jnp.int32(1))
    pltpu.async_copy(m_hbm.at[idx_row(0)], buf0, sem0)
    pltpu.async_copy(m_hbm.at[idx_row(1)], buf1, sem1)

    def pair_body(h, _):
        q0 = h * 2

        @pl.when(q0 + 2 < qpw)
        def _():
            compact_row(q0 + 2)
            compact_row(q0 + 3)

        pltpu.make_async_copy(m_hbm.at[idx_row(0)], buf0, sem0).wait()
        accumulate(buf0, q0)

        @pl.when(q0 + 2 < qpw)
        def _():
            pltpu.async_copy(m_hbm.at[idx_row(q0 + 2)], buf0, sem0)

        pltpu.make_async_copy(m_hbm.at[idx_row(0)], buf1, sem1).wait()
        accumulate(buf1, q0 + 1)

        @pl.when(q0 + 3 < qpw)
        def _():
            pltpu.async_copy(m_hbm.at[idx_row(q0 + 3)], buf1, sem1)

        return 0

    lax.fori_loop(0, qpw // 2, pair_body, 0)

    pltpu.sync_copy(s1_v, s1_hbm.at[pl.ds(base, qpw)])
    pltpu.sync_copy(s2_v, s2_hbm.at[pl.ds(base, qpw)])


def _run_sc_gather(selidx, m, nrows):
    qpw = nrows // NW
    mesh = plsc.VectorSubcoreMesh(core_axis_name="c", subcore_axis_name="s")
    f = pl.kernel(
        functools.partial(_sc_gather_kernel, qpw),
        mesh=mesh,
        compiler_params=pltpu.CompilerParams(needs_layout_passes=False),
        out_type=(
            jax.ShapeDtypeStruct((nrows, D), jnp.float32),
            jax.ShapeDtypeStruct((nrows, D), jnp.float32),
        ),
        scratch_types=[
            pltpu.VMEM((qpw, B), jnp.int32),
            pltpu.VMEM((qpw, K + L), jnp.int32),
            pltpu.VMEM((K, D), jnp.float32),
            pltpu.VMEM((K, D), jnp.float32),
            pltpu.VMEM((qpw, D), jnp.float32),
            pltpu.VMEM((qpw, D), jnp.float32),
            pltpu.SemaphoreType.DMA,
            pltpu.SemaphoreType.DMA,
        ],
    )
    return f(selidx, m)


def _loss_kernel(s1a_ref, s2a_ref, s1b_ref, s2b_ref, v_ref, out_ref):
    v = v_ref[...]
    HB = B // 2

    def half(s1, s2, vh):
        var = (s2 - s1 * s1 * (1.0 / K)) * (1.0 / (K - 1))
        nstd = jnp.sqrt(jnp.maximum(var, 0.0))
        pstd = jnp.exp(vh * 0.5)
        return jnp.sum((pstd - nstd) ** 2)

    out_ref[0, 0] = half(
        s1a_ref[...], s2a_ref[...], lax.slice(v, (0, 0), (HB, D))
    ) + half(
        s1b_ref[...], s2b_ref[...], lax.slice(v, (HB, 0), (B, D))
    )


def _run_loss(s1a, s2a, s1b, s2b, v):
    return pl.pallas_call(
        _loss_kernel,
        out_specs=pl.BlockSpec(memory_space=pltpu.SMEM),
        out_shape=jax.ShapeDtypeStruct((1, 1), jnp.float32),
    )(s1a, s2a, s1b, s2b, v)


def kernel(input1_mean, input1_var, input2_mean, input2_var):
    del input2_mean, input2_var
    HB = B // 2
    selidx_a = _run_select(input1_mean, 0, HB)
    s1a, s2a = _run_sc_gather(selidx_a, input1_mean, HB)
    selidx_b = _run_select(input1_mean, HB, HB)
    s1b, s2b = _run_sc_gather(selidx_b, input1_mean, HB)
    out = _run_loss(s1a, s2a, s1b, s2b, input1_var)
    return (out[0, 0] * (1.0 / (B * D))).reshape(())
